# Initial kernel scaffold; baseline (speedup 1.0000x reference)
import jax
import jax.numpy as jnp
from jax.experimental import pallas as pl


def kernel(x):
    raise NotImplementedError("write your pallas kernel here")


# speedup vs baseline: 1.0074x; 1.0074x over previous
import jax
import jax.numpy as jnp
from jax.experimental import pallas as pl


def _copy_body(x_ref, o_ref):
    o_ref[...] = x_ref[...]


def kernel(x):
    b, s, d = x.shape
    x2 = x.reshape(s, d).astype(jnp.float32)
    out = pl.pallas_call(
        _copy_body,
        out_shape=jax.ShapeDtypeStruct((s, d), jnp.float32),
    )(x2)
    return out.reshape(b, s, d)
